# Initial kernel scaffold; baseline (speedup 1.0000x reference)
import jax
import jax.numpy as jnp
from jax.experimental import pallas as pl


def kernel(xyz, features, neighbors_idx, W1, b1, gamma, beta, W2, b2):
    raise NotImplementedError("write your pallas kernel here")


# speedup vs baseline: 26.7204x; 26.7204x over previous
import functools

import jax
import jax.numpy as jnp
from jax import lax
from jax.experimental import pallas as pl
from jax.experimental.pallas import tpu as pltpu
from jax.experimental.pallas import tpu_sc as plsc

BN_EPS = 1e-5

CHUNK = 80
NCHUNK = 250
NBUF = 10
LOOKAHEAD = 5


def _precompute_body(feats_ref, xyz_ref, wx_ref, wc_ref, wn_ref, b1_ref,
                     g_ref, hc_ref):
    p = jnp.dot(xyz_ref[...], wx_ref[...], preferred_element_type=jnp.float32)
    f = feats_ref[...]
    g_ref[...] = jnp.dot(f, wn_ref[...], preferred_element_type=jnp.float32) + p
    hc_ref[...] = (jnp.dot(f, wc_ref[...], preferred_element_type=jnp.float32)
                   - p + b1_ref[...])


def _precompute(feats, xyz8, wx, wc, wn, b1r, rows, blk):
    nb = rows // blk
    return pl.pallas_call(
        _precompute_body,
        grid=(nb,),
        in_specs=[
            pl.BlockSpec((blk, 64), lambda i: (i, 0)),
            pl.BlockSpec((blk, 8), lambda i: (i, 0)),
            pl.BlockSpec((8, 64), lambda i: (0, 0)),
            pl.BlockSpec((64, 64), lambda i: (0, 0)),
            pl.BlockSpec((64, 64), lambda i: (0, 0)),
            pl.BlockSpec((1, 64), lambda i: (0, 0)),
        ],
        out_specs=[
            pl.BlockSpec((blk, 64), lambda i: (i, 0)),
            pl.BlockSpec((blk, 64), lambda i: (i, 0)),
        ],
        out_shape=[
            jax.ShapeDtypeStruct((rows, 64), jnp.float32),
            jax.ShapeDtypeStruct((rows, 64), jnp.float32),
        ],
    )(feats, xyz8, wx, wc, wn, b1r)


def _sc_gather_body(gflat_hbm, idxt_hbm, out_hbm, idx_v, buf_v, gsem, ssem):
    h = lax.axis_index("c")
    k = lax.axis_index("s")
    base = h * (NCHUNK * CHUNK)

    pltpu.sync_copy(idxt_hbm.at[k, h], idx_v)

    def gather_start(ch, b):
        pltpu.async_copy(gflat_hbm.at[idx_v.at[ch]], buf_v.at[b], gsem)

    def gather_wait(ch, b):
        pltpu.make_async_copy(gflat_hbm.at[idx_v.at[ch]], buf_v.at[b],
                              gsem).wait()

    def store_start(ch, b):
        pltpu.async_copy(buf_v.at[b],
                         out_hbm.at[k, pl.ds(base + ch * CHUNK, CHUNK)], ssem)

    def store_wait(ch, b):
        pltpu.make_async_copy(buf_v.at[b],
                              out_hbm.at[k, pl.ds(base + ch * CHUNK, CHUNK)],
                              ssem).wait()

    ngroups = NCHUNK // NBUF

    for j in range(LOOKAHEAD):
        gather_start(j, j)

    for j in range(NBUF):
        gather_wait(j, j)
        store_start(j, j)
        if j >= LOOKAHEAD:
            store_wait(j - LOOKAHEAD, (j + LOOKAHEAD) % NBUF)
        gather_start(j + LOOKAHEAD, (j + LOOKAHEAD) % NBUF)

    def steady(g, carry):
        ch0 = g * NBUF
        for j in range(NBUF):
            ch = ch0 + j
            gather_wait(ch, j)
            store_start(ch, j)
            store_wait(ch + LOOKAHEAD - NBUF, (j + LOOKAHEAD) % NBUF)
            gather_start(ch + LOOKAHEAD, (j + LOOKAHEAD) % NBUF)
        return carry

    lax.fori_loop(1, ngroups - 1, steady, 0)

    for j in range(NBUF):
        ch = NCHUNK - NBUF + j
        gather_wait(ch, j)
        store_start(ch, j)
        if j < LOOKAHEAD:
            store_wait(ch + LOOKAHEAD - NBUF, (j + LOOKAHEAD) % NBUF)
            gather_start(ch + LOOKAHEAD, (j + LOOKAHEAD) % NBUF)

    for j in range(NBUF):
        ch = NCHUNK - NBUF + j
        store_wait(ch, j)


def _sc_gather(gflat, idxt, k_planes, rows):
    mesh = plsc.VectorSubcoreMesh(core_axis_name="c", subcore_axis_name="s")
    fn = pl.kernel(
        _sc_gather_body,
        out_type=jax.ShapeDtypeStruct((k_planes, rows, 64), jnp.float32),
        mesh=mesh,
        compiler_params=pltpu.CompilerParams(use_tc_tiling_on_sc=False),
        scratch_types=[
            pltpu.VMEM((NCHUNK, CHUNK), jnp.int32),
            pltpu.VMEM((NBUF, CHUNK, 64), jnp.float32),
            pltpu.SemaphoreType.DMA,
            pltpu.SemaphoreType.DMA,
        ],
    )
    return fn(gflat, idxt)


def _stats_body(gg_ref, hc_ref, sum_ref, ssq_ref, acc_s, acc_q):
    k = pl.program_id(0)
    pb = pl.program_id(1)
    x = gg_ref[0] + hc_ref[...]
    bs = jnp.sum(x, axis=0, keepdims=True)
    bq = jnp.sum(x * x, axis=0, keepdims=True)

    @pl.when(jnp.logical_and(k == 0, pb == 0))
    def _init():
        acc_s[...] = bs
        acc_q[...] = bq

    @pl.when(jnp.logical_not(jnp.logical_and(k == 0, pb == 0)))
    def _acc():
        acc_s[...] += bs
        acc_q[...] += bq

    @pl.when(jnp.logical_and(k == pl.num_programs(0) - 1,
                             pb == pl.num_programs(1) - 1))
    def _fin():
        sum_ref[...] = acc_s[...]
        ssq_ref[...] = acc_q[...]


def _stats(gg, hc, k_planes, rows, blk):
    nb = rows // blk
    return pl.pallas_call(
        _stats_body,
        grid=(k_planes, nb),
        in_specs=[
            pl.BlockSpec((1, blk, 64), lambda k, pb: (k, pb, 0)),
            pl.BlockSpec((blk, 64), lambda k, pb: (pb, 0)),
        ],
        out_specs=[
            pl.BlockSpec((1, 64), lambda k, pb: (0, 0)),
            pl.BlockSpec((1, 64), lambda k, pb: (0, 0)),
        ],
        out_shape=[
            jax.ShapeDtypeStruct((1, 64), jnp.float32),
            jax.ShapeDtypeStruct((1, 64), jnp.float32),
        ],
        scratch_shapes=[
            pltpu.VMEM((1, 64), jnp.float32),
            pltpu.VMEM((1, 64), jnp.float32),
        ],
    )(gg, hc)


def _apply_body(gg_ref, hc_ref, sum_ref, ssq_ref, gam_ref, bet_ref, w2_ref,
                b2_ref, out_ref, macc, *, m_total):
    k = pl.program_id(1)
    mean = sum_ref[...] * (1.0 / m_total)
    var = ssq_ref[...] * (1.0 / m_total) - mean * mean
    s = gam_ref[...] * lax.rsqrt(var + BN_EPS)
    t = bet_ref[...] - mean * s

    x = gg_ref[0] + hc_ref[...]
    y = jnp.maximum(x * s + t, 0.0)
    z = jnp.dot(y, w2_ref[...], preferred_element_type=jnp.float32)

    @pl.when(k == 0)
    def _init():
        macc[...] = z

    @pl.when(k != 0)
    def _max():
        macc[...] = jnp.maximum(macc[...], z)

    @pl.when(k == pl.num_programs(1) - 1)
    def _fin():
        out_ref[...] = macc[...] + b2_ref[...]


def _apply(gg, hc, ssum, ssq, gam, bet, w2, b2r, k_planes, rows, blk):
    nb = rows // blk
    m_total = float(k_planes * rows)
    return pl.pallas_call(
        functools.partial(_apply_body, m_total=m_total),
        grid=(nb, k_planes),
        in_specs=[
            pl.BlockSpec((1, blk, 64), lambda pb, k: (k, pb, 0)),
            pl.BlockSpec((blk, 64), lambda pb, k: (pb, 0)),
            pl.BlockSpec((1, 64), lambda pb, k: (0, 0)),
            pl.BlockSpec((1, 64), lambda pb, k: (0, 0)),
            pl.BlockSpec((1, 64), lambda pb, k: (0, 0)),
            pl.BlockSpec((1, 64), lambda pb, k: (0, 0)),
            pl.BlockSpec((64, 64), lambda pb, k: (0, 0)),
            pl.BlockSpec((1, 64), lambda pb, k: (0, 0)),
        ],
        out_specs=pl.BlockSpec((blk, 64), lambda pb, k: (pb, 0)),
        out_shape=jax.ShapeDtypeStruct((rows, 64), jnp.float32),
        scratch_shapes=[pltpu.VMEM((blk, 64), jnp.float32)],
    )(gg, hc, ssum, ssq, gam, bet, w2, b2r)


def kernel(xyz, features, neighbors_idx, W1, b1, gamma, beta, W2, b2):
    B, N, C = features.shape
    K = neighbors_idx.shape[-1]
    rows = B * N
    blk = 4000

    feats = features.reshape(rows, C)
    xyz8 = jnp.zeros((rows, 8), jnp.float32).at[:, :3].set(xyz.reshape(rows, 3))
    wx = jnp.zeros((8, 64), jnp.float32).at[:3].set(W1[0:3])
    wc = W1[3:3 + C]
    wn = W1[3 + C:3 + 2 * C]
    b1r = b1.reshape(1, 64)

    idx = neighbors_idx.astype(jnp.int32) + (jnp.arange(B, dtype=jnp.int32)
                                             * N)[:, None, None]
    idxt = idx.transpose(2, 0, 1).reshape(K, 2, NCHUNK, CHUNK)

    g, hc = _precompute(feats, xyz8, wx, wc, wn, b1r, rows, blk)
    gg = _sc_gather(g, idxt, K, rows)
    ssum, ssq = _stats(gg, hc, K, rows, blk)
    out = _apply(gg, hc, ssum, ssq, gamma.reshape(1, 64), beta.reshape(1, 64),
                 W2, b2.reshape(1, 64), K, rows, blk)
    return out.reshape(B, N, 64)
